# Initial kernel scaffold; baseline (speedup 1.0000x reference)
import jax
import jax.numpy as jnp
from jax.experimental import pallas as pl


def kernel(x, edge_index, W1, att_src1, att_dst1, b1, W2, att_src2, att_dst2, b2):
    raise NotImplementedError("write your pallas kernel here")


# speedup vs baseline: 40.3381x; 40.3381x over previous
import jax
import jax.numpy as jnp
from jax import lax
from jax.experimental import pallas as pl
from jax.experimental.pallas import tpu as pltpu
from jax.experimental.pallas import tpu_sc as plsc

_N = 10000
_E = 320000
_F = 128
_NH = 8
_NC = 16
_ALPHA = 0.5

_NSUB = 16
_LANES = 16
_EPT = _E // _NSUB
_B = 80
_NIT = _EPT // _B
_ZR = 624
_ZTAIL = _N - _NSUB * _ZR
_RB = 2000


def _pre_body(x_ref, w_ref, as_ref, ad_ref, h_ref, s_ref, d_ref, m_ref):
    xv = x_ref[...]
    h = lax.dot_general(xv, w_ref[0], (((1,), (1,)), ((), ())),
                        preferred_element_type=jnp.float32)
    s = jnp.dot(h, as_ref[0], preferred_element_type=jnp.float32)
    d = jnp.dot(h, ad_ref[0], preferred_element_type=jnp.float32)
    h_ref[0] = h
    s_ref[0] = s
    d_ref[0] = d
    b = jnp.max(s, axis=0, keepdims=True) + jnp.max(d, axis=0, keepdims=True)
    m = jnp.maximum(b, 0.2 * b)
    m_ref[0] = jnp.concatenate([m, m], axis=1)


def _sc_body(srcg_hbm, dstg_hbm, dsts_hbm, s_hbm, d_hbm, h_hbm, m_hbm,
             z128_hbm, z16_hbm, out_hbm, den_hbm,
             srcg_v, dstg_v, dsts_v, sg, dg, w2w, hrows, mvec_v,
             outacc, denacc, sem):
    c = lax.axis_index("c")
    t = lax.axis_index("s")
    rbase = t * _ZR

    pltpu.sync_copy(z128_hbm, outacc.at[pl.ds(rbase, _ZR)])
    pltpu.sync_copy(z16_hbm, denacc.at[pl.ds(rbase, _ZR)])

    @pl.when(t == _NSUB - 1)
    def _zero_tail():
        pltpu.sync_copy(z128_hbm.at[pl.ds(0, _ZTAIL)],
                        outacc.at[pl.ds(_NSUB * _ZR, _ZTAIL)])
        pltpu.sync_copy(z16_hbm.at[pl.ds(0, _ZTAIL)],
                        denacc.at[pl.ds(_NSUB * _ZR, _ZTAIL)])

    zv = jnp.zeros((_LANES,), jnp.float32)
    for i in range(_B):
        w2w[i, :] = zv

    pltpu.sync_copy(m_hbm.at[pl.ds(c * 16, 16)], mvec_v)
    plsc.subcore_barrier()

    ebase0 = c * _E + t * _EPT

    def iteration(i, carry):
        eb = ebase0 + i * _B
        pltpu.sync_copy(srcg_hbm.at[pl.ds(eb, _B)], srcg_v)
        pltpu.sync_copy(dstg_hbm.at[pl.ds(eb, _B)], dstg_v)
        pltpu.sync_copy(dsts_hbm.at[pl.ds(eb, _B)], dsts_v)
        pltpu.async_copy(s_hbm.at[srcg_v], sg, sem).wait()
        pltpu.async_copy(d_hbm.at[dstg_v], dg, sem).wait()
        pltpu.async_copy(h_hbm.at[srcg_v], hrows, sem).wait()

        def wbody(k, carry2):
            lanes = lax.iota(jnp.int32, 16)
            l8 = lanes // 8
            cols = lanes - 8 * l8
            mv = mvec_v[...]
            rows = 2 * k + l8
            g = (plsc.load_gather(sg, [rows, cols])
                 + plsc.load_gather(dg, [rows, cols]))
            wv = jnp.exp(jnp.maximum(g, 0.2 * g) - mv)
            plsc.store_scatter(w2w, [rows, cols], wv)
            return carry2

        lax.fori_loop(0, _B // 2, wbody, 0)

        def mbody(e, carry2):
            ev = jnp.full((_LANES,), e, dtype=jnp.int32)
            for hh in range(_NH):
                hv = jnp.full((_LANES,), hh, dtype=jnp.int32)
                wb = plsc.load_gather(w2w, [ev, hv])
                hrows[e, pl.ds(hh * 16, 16)] = hrows[e, pl.ds(hh * 16, 16)] * wb
            return carry2

        lax.fori_loop(0, _B, mbody, 0)

        pltpu.sync_copy(hrows, outacc.at[dsts_v], add=True)
        pltpu.sync_copy(w2w, denacc.at[dsts_v], add=True)
        return carry

    lax.fori_loop(0, _NIT, iteration, 0)

    plsc.subcore_barrier()
    pltpu.sync_copy(outacc.at[pl.ds(rbase, _ZR)],
                    out_hbm.at[pl.ds(c * _N + rbase, _ZR)])
    pltpu.sync_copy(denacc.at[pl.ds(rbase, _ZR)],
                    den_hbm.at[pl.ds(c * _N + rbase, _ZR)])

    @pl.when(t == _NSUB - 1)
    def _write_tail():
        pltpu.sync_copy(outacc.at[pl.ds(_NSUB * _ZR, _ZTAIL)],
                        out_hbm.at[pl.ds(c * _N + _NSUB * _ZR, _ZTAIL)])
        pltpu.sync_copy(denacc.at[pl.ds(_NSUB * _ZR, _ZTAIL)],
                        den_hbm.at[pl.ds(c * _N + _NSUB * _ZR, _ZTAIL)])


def _post_body(u_ref, den_ref, h_ref, s_ref, d_ref, m_ref, b_ref, e_ref,
               out_ref):
    acc = None
    for dd in range(2):
        g = s_ref[dd] + d_ref[dd]
        m8 = m_ref[dd, 0:1, 0:8]
        ws = jnp.exp(jnp.maximum(g, 0.2 * g) - m8)
        wfull = jnp.dot(ws, e_ref[...], preferred_element_type=jnp.float32)
        dfull = jnp.dot(den_ref[dd, :, 0:8] + ws, e_ref[...],
                        preferred_element_type=jnp.float32)
        od = ((u_ref[dd] + wfull * h_ref[dd]) / (dfull + 1e-16)
              + b_ref[dd:dd + 1, :])
        w = (1.0 - _ALPHA) if dd == 0 else _ALPHA
        acc = w * od if acc is None else acc + w * od
    out_ref[...] = acc


def kernel(x, edge_index, W1, att_src1, att_dst1, b1, W2, att_src2,
           att_dst2, b2):
    src = edge_index[0]
    dst = edge_index[1]
    srcg = jnp.concatenate([src, dst + _N])
    dstg = jnp.concatenate([dst, src + _N])
    dsts = jnp.concatenate([dst, src])

    def blockdiag(att):
        a = att.reshape(_NH, _NC)
        return (jnp.eye(_NH, dtype=a.dtype)[:, None, :]
                * a[:, :, None]).reshape(_NH * _NC, _NH)

    Wstk = jnp.stack([W1, W2])
    AS = jnp.stack([blockdiag(att_src1), blockdiag(att_src2)])
    AD = jnp.stack([blockdiag(att_dst1), blockdiag(att_dst2)])
    Emat = jnp.repeat(jnp.eye(_NH, dtype=jnp.float32), _NC, axis=1)

    Hc, Sc, Dc, Mc = pl.pallas_call(
        _pre_body,
        grid=(2,),
        in_specs=[
            pl.BlockSpec((_N, _F), lambda d: (0, 0)),
            pl.BlockSpec((1, _F, _F), lambda d: (d, 0, 0)),
            pl.BlockSpec((1, _F, _NH), lambda d: (d, 0, 0)),
            pl.BlockSpec((1, _F, _NH), lambda d: (d, 0, 0)),
        ],
        out_specs=[
            pl.BlockSpec((1, _N, _F), lambda d: (d, 0, 0)),
            pl.BlockSpec((1, _N, _NH), lambda d: (d, 0, 0)),
            pl.BlockSpec((1, _N, _NH), lambda d: (d, 0, 0)),
            pl.BlockSpec((1, 1, 16), lambda d: (d, 0, 0)),
        ],
        out_shape=[
            jax.ShapeDtypeStruct((2, _N, _F), jnp.float32),
            jax.ShapeDtypeStruct((2, _N, _NH), jnp.float32),
            jax.ShapeDtypeStruct((2, _N, _NH), jnp.float32),
            jax.ShapeDtypeStruct((2, 1, 16), jnp.float32),
        ],
    )(x, Wstk, AS, AD)

    mesh = plsc.VectorSubcoreMesh(core_axis_name="c", subcore_axis_name="s")
    outu, den = pl.kernel(
        _sc_body,
        out_type=[
            jax.ShapeDtypeStruct((2 * _N, _F), jnp.float32),
            jax.ShapeDtypeStruct((2 * _N, 16), jnp.float32),
        ],
        mesh=mesh,
        compiler_params=pltpu.CompilerParams(use_tc_tiling_on_sc=False,
                                             needs_layout_passes=False),
        scratch_types=[
            pltpu.VMEM((_B,), jnp.int32),
            pltpu.VMEM((_B,), jnp.int32),
            pltpu.VMEM((_B,), jnp.int32),
            pltpu.VMEM((_B, _NH), jnp.float32),
            pltpu.VMEM((_B, _NH), jnp.float32),
            pltpu.VMEM((_B, 16), jnp.float32),
            pltpu.VMEM((_B, _F), jnp.float32),
            pltpu.VMEM((16,), jnp.float32),
            pltpu.VMEM_SHARED((_N, _F), jnp.float32),
            pltpu.VMEM_SHARED((_N, 16), jnp.float32),
            pltpu.SemaphoreType.DMA,
        ],
    )(srcg, dstg, dsts,
      Sc.reshape(2 * _N, _NH), Dc.reshape(2 * _N, _NH),
      Hc.reshape(2 * _N, _F), Mc.reshape(32),
      jnp.zeros((_ZR, _F), jnp.float32), jnp.zeros((_ZR, 16), jnp.float32))

    out = pl.pallas_call(
        _post_body,
        grid=(_N // _RB,),
        in_specs=[
            pl.BlockSpec((2, _RB, _F), lambda i: (0, i, 0)),
            pl.BlockSpec((2, _RB, 16), lambda i: (0, i, 0)),
            pl.BlockSpec((2, _RB, _F), lambda i: (0, i, 0)),
            pl.BlockSpec((2, _RB, _NH), lambda i: (0, i, 0)),
            pl.BlockSpec((2, _RB, _NH), lambda i: (0, i, 0)),
            pl.BlockSpec((2, 1, 16), lambda i: (0, 0, 0)),
            pl.BlockSpec((2, _F), lambda i: (0, 0)),
            pl.BlockSpec((_NH, _F), lambda i: (0, 0)),
        ],
        out_specs=pl.BlockSpec((_RB, _F), lambda i: (i, 0)),
        out_shape=jax.ShapeDtypeStruct((_N, _F), jnp.float32),
    )(outu.reshape(2, _N, _F), den.reshape(2, _N, 16), Hc, Sc, Dc, Mc,
      jnp.stack([b1, b2]), Emat)
    return out


# speedup vs baseline: 143.7274x; 3.5631x over previous
import jax
import jax.numpy as jnp
from jax import lax
from jax.experimental import pallas as pl
from jax.experimental.pallas import tpu as pltpu
from jax.experimental.pallas import tpu_sc as plsc

_N = 10000
_E = 320000
_F = 128
_NH = 8
_NC = 16
_ALPHA = 0.5

_NSUB = 16
_LANES = 16
_EPT = _E // _NSUB
_B = 80
_NIT = _EPT // _B
_ZR = 624
_ZTAIL = _N - _NSUB * _ZR
_RB = 2000


def _pre_body(x_ref, w_ref, as_ref, ad_ref, h_ref, s_ref, d_ref, m_ref):
    xv = x_ref[...]
    h = lax.dot_general(xv, w_ref[0], (((1,), (1,)), ((), ())),
                        preferred_element_type=jnp.float32)
    s = jnp.dot(h, as_ref[0], preferred_element_type=jnp.float32)
    d = jnp.dot(h, ad_ref[0], preferred_element_type=jnp.float32)
    h_ref[0] = h
    s_ref[0] = s
    d_ref[0] = d
    b = jnp.max(s, axis=0, keepdims=True) + jnp.max(d, axis=0, keepdims=True)
    m = jnp.maximum(b, 0.2 * b)
    m_ref[0] = jnp.concatenate([m, m], axis=1)


def _sc_body(srcg_hbm, dstg_hbm, dsts_hbm, s_hbm, d_hbm, h_hbm, m_hbm,
             z128_hbm, z16_hbm, out_hbm, den_hbm,
             srcg_v0, srcg_v1, dstg_v0, dstg_v1, dsts_v0, dsts_v1,
             sg0, sg1, dg0, dg1, w2w0, w2w1, hrows0, hrows1, mvec_v,
             outacc, denacc,
             sem_i0, sem_i1, sem_g0, sem_g1, sem_s0, sem_s1):
    srcg_v = (srcg_v0, srcg_v1)
    dstg_v = (dstg_v0, dstg_v1)
    dsts_v = (dsts_v0, dsts_v1)
    sg = (sg0, sg1)
    dg = (dg0, dg1)
    w2w = (w2w0, w2w1)
    hrows = (hrows0, hrows1)
    sem_i = (sem_i0, sem_i1)
    sem_g = (sem_g0, sem_g1)
    sem_s = (sem_s0, sem_s1)

    c = lax.axis_index("c")
    t = lax.axis_index("s")
    rbase = t * _ZR

    pltpu.sync_copy(z128_hbm, outacc.at[pl.ds(rbase, _ZR)])
    pltpu.sync_copy(z16_hbm, denacc.at[pl.ds(rbase, _ZR)])

    @pl.when(t == _NSUB - 1)
    def _zero_tail():
        pltpu.sync_copy(z128_hbm.at[pl.ds(0, _ZTAIL)],
                        outacc.at[pl.ds(_NSUB * _ZR, _ZTAIL)])
        pltpu.sync_copy(z16_hbm.at[pl.ds(0, _ZTAIL)],
                        denacc.at[pl.ds(_NSUB * _ZR, _ZTAIL)])

    zv = jnp.zeros((_LANES,), jnp.float32)
    for bb in range(2):
        for i in range(_B):
            w2w[bb][i, :] = zv

    pltpu.sync_copy(m_hbm.at[pl.ds(c * 16, 16)], mvec_v)
    plsc.subcore_barrier()

    ebase0 = c * _E + t * _EPT

    def fire_idx(blk, b):
        eb = ebase0 + blk * _B
        pltpu.async_copy(srcg_hbm.at[pl.ds(eb, _B)], srcg_v[b], sem_i[b])
        pltpu.async_copy(dstg_hbm.at[pl.ds(eb, _B)], dstg_v[b], sem_i[b])
        pltpu.async_copy(dsts_hbm.at[pl.ds(eb, _B)], dsts_v[b], sem_i[b])

    def wait_idx(b):
        pltpu.make_async_copy(srcg_hbm.at[pl.ds(0, _B)], srcg_v[b], sem_i[b]).wait()
        pltpu.make_async_copy(dstg_hbm.at[pl.ds(0, _B)], dstg_v[b], sem_i[b]).wait()
        pltpu.make_async_copy(dsts_hbm.at[pl.ds(0, _B)], dsts_v[b], sem_i[b]).wait()

    def fire_gathers(b):
        pltpu.async_copy(s_hbm.at[srcg_v[b]], sg[b], sem_g[b])
        pltpu.async_copy(d_hbm.at[dstg_v[b]], dg[b], sem_g[b])
        pltpu.async_copy(h_hbm.at[srcg_v[b]], hrows[b], sem_g[b])

    def wait_gathers(b):
        pltpu.make_async_copy(s_hbm.at[srcg_v[b]], sg[b], sem_g[b]).wait()
        pltpu.make_async_copy(d_hbm.at[dstg_v[b]], dg[b], sem_g[b]).wait()
        pltpu.make_async_copy(h_hbm.at[srcg_v[b]], hrows[b], sem_g[b]).wait()

    def fire_scatter(b):
        pltpu.async_copy(hrows[b], outacc.at[dsts_v[b]], sem_s[b], add=True)
        pltpu.async_copy(w2w[b], denacc.at[dsts_v[b]], sem_s[b], add=True)

    def wait_scatter(b):
        pltpu.make_async_copy(hrows[b], outacc.at[dsts_v[b]], sem_s[b]).wait()
        pltpu.make_async_copy(w2w[b], denacc.at[dsts_v[b]], sem_s[b]).wait()

    def compute(b):
        def wbody(k, carry2):
            lanes = lax.iota(jnp.int32, 16)
            l8 = lanes // 8
            cols = lanes - 8 * l8
            mv = mvec_v[...]
            rows = 2 * k + l8
            g = (plsc.load_gather(sg[b], [rows, cols])
                 + plsc.load_gather(dg[b], [rows, cols]))
            wv = jnp.exp(jnp.maximum(g, 0.2 * g) - mv)
            plsc.store_scatter(w2w[b], [rows, cols], wv)
            return carry2

        lax.fori_loop(0, _B // 2, wbody, 0)

        def mbody(e, carry2):
            wrow = w2w[b][e, :]
            for hh in range(_NH):
                hv = jnp.full((_LANES,), hh, dtype=jnp.int32)
                wb = wrow.at[hv].get(mode="promise_in_bounds")
                hrows[b][e, pl.ds(hh * 16, 16)] = (
                    hrows[b][e, pl.ds(hh * 16, 16)] * wb)
            return carry2

        lax.fori_loop(0, _B, mbody, 0)

    pltpu.sync_copy(srcg_hbm.at[pl.ds(ebase0, _B)], srcg_v[0])
    pltpu.sync_copy(dstg_hbm.at[pl.ds(ebase0, _B)], dstg_v[0])
    pltpu.sync_copy(dsts_hbm.at[pl.ds(ebase0, _B)], dsts_v[0])
    fire_gathers(0)
    fire_idx(1, 1)

    def phase(blk, b):
        nb = 1 - b

        @pl.when(blk + 1 < _NIT)
        def _prefetch():
            wait_idx(nb)

            @pl.when(blk >= 1)
            def _protect():
                wait_scatter(nb)

            fire_gathers(nb)

        wait_gathers(b)
        compute(b)
        fire_scatter(b)

        @pl.when(blk + 2 < _NIT)
        def _next_idx():
            fire_idx(blk + 2, b)

    def outer(g, carry):
        phase(2 * g, 0)
        phase(2 * g + 1, 1)
        return carry

    lax.fori_loop(0, _NIT // 2, outer, 0)

    wait_scatter(0)
    wait_scatter(1)
    plsc.subcore_barrier()
    pltpu.sync_copy(outacc.at[pl.ds(rbase, _ZR)],
                    out_hbm.at[pl.ds(c * _N + rbase, _ZR)])
    pltpu.sync_copy(denacc.at[pl.ds(rbase, _ZR)],
                    den_hbm.at[pl.ds(c * _N + rbase, _ZR)])

    @pl.when(t == _NSUB - 1)
    def _write_tail():
        pltpu.sync_copy(outacc.at[pl.ds(_NSUB * _ZR, _ZTAIL)],
                        out_hbm.at[pl.ds(c * _N + _NSUB * _ZR, _ZTAIL)])
        pltpu.sync_copy(denacc.at[pl.ds(_NSUB * _ZR, _ZTAIL)],
                        den_hbm.at[pl.ds(c * _N + _NSUB * _ZR, _ZTAIL)])


def _post_body(u_ref, den_ref, h_ref, s_ref, d_ref, m_ref, b_ref, e_ref,
               out_ref):
    acc = None
    for dd in range(2):
        g = s_ref[dd] + d_ref[dd]
        m8 = m_ref[dd, 0:1, 0:8]
        ws = jnp.exp(jnp.maximum(g, 0.2 * g) - m8)
        wfull = jnp.dot(ws, e_ref[...], preferred_element_type=jnp.float32)
        dfull = jnp.dot(den_ref[dd, :, 0:8] + ws, e_ref[...],
                        preferred_element_type=jnp.float32)
        od = ((u_ref[dd] + wfull * h_ref[dd]) / (dfull + 1e-16)
              + b_ref[dd:dd + 1, :])
        w = (1.0 - _ALPHA) if dd == 0 else _ALPHA
        acc = w * od if acc is None else acc + w * od
    out_ref[...] = acc


def kernel(x, edge_index, W1, att_src1, att_dst1, b1, W2, att_src2,
           att_dst2, b2):
    src = edge_index[0]
    dst = edge_index[1]
    srcg = jnp.concatenate([src, dst + _N])
    dstg = jnp.concatenate([dst, src + _N])
    dsts = jnp.concatenate([dst, src])

    def blockdiag(att):
        a = att.reshape(_NH, _NC)
        return (jnp.eye(_NH, dtype=a.dtype)[:, None, :]
                * a[:, :, None]).reshape(_NH * _NC, _NH)

    Wstk = jnp.stack([W1, W2])
    AS = jnp.stack([blockdiag(att_src1), blockdiag(att_src2)])
    AD = jnp.stack([blockdiag(att_dst1), blockdiag(att_dst2)])
    Emat = jnp.repeat(jnp.eye(_NH, dtype=jnp.float32), _NC, axis=1)

    Hc, Sc, Dc, Mc = pl.pallas_call(
        _pre_body,
        grid=(2,),
        in_specs=[
            pl.BlockSpec((_N, _F), lambda d: (0, 0)),
            pl.BlockSpec((1, _F, _F), lambda d: (d, 0, 0)),
            pl.BlockSpec((1, _F, _NH), lambda d: (d, 0, 0)),
            pl.BlockSpec((1, _F, _NH), lambda d: (d, 0, 0)),
        ],
        out_specs=[
            pl.BlockSpec((1, _N, _F), lambda d: (d, 0, 0)),
            pl.BlockSpec((1, _N, _NH), lambda d: (d, 0, 0)),
            pl.BlockSpec((1, _N, _NH), lambda d: (d, 0, 0)),
            pl.BlockSpec((1, 1, 16), lambda d: (d, 0, 0)),
        ],
        out_shape=[
            jax.ShapeDtypeStruct((2, _N, _F), jnp.float32),
            jax.ShapeDtypeStruct((2, _N, _NH), jnp.float32),
            jax.ShapeDtypeStruct((2, _N, _NH), jnp.float32),
            jax.ShapeDtypeStruct((2, 1, 16), jnp.float32),
        ],
    )(x, Wstk, AS, AD)

    mesh = plsc.VectorSubcoreMesh(core_axis_name="c", subcore_axis_name="s")
    outu, den = pl.kernel(
        _sc_body,
        out_type=[
            jax.ShapeDtypeStruct((2 * _N, _F), jnp.float32),
            jax.ShapeDtypeStruct((2 * _N, 16), jnp.float32),
        ],
        mesh=mesh,
        compiler_params=pltpu.CompilerParams(use_tc_tiling_on_sc=False,
                                             needs_layout_passes=False),
        scratch_types=(
            [pltpu.VMEM((_B,), jnp.int32)] * 6
            + [pltpu.VMEM((_B, _NH), jnp.float32)] * 4
            + [pltpu.VMEM((_B, 16), jnp.float32)] * 2
            + [pltpu.VMEM((_B, _F), jnp.float32)] * 2
            + [pltpu.VMEM((16,), jnp.float32)]
            + [pltpu.VMEM_SHARED((_N, _F), jnp.float32),
               pltpu.VMEM_SHARED((_N, 16), jnp.float32)]
            + [pltpu.SemaphoreType.DMA] * 6
        ),
    )(srcg, dstg, dsts,
      Sc.reshape(2 * _N, _NH), Dc.reshape(2 * _N, _NH),
      Hc.reshape(2 * _N, _F), Mc.reshape(32),
      jnp.zeros((_ZR, _F), jnp.float32), jnp.zeros((_ZR, 16), jnp.float32))

    out = pl.pallas_call(
        _post_body,
        grid=(_N // _RB,),
        in_specs=[
            pl.BlockSpec((2, _RB, _F), lambda i: (0, i, 0)),
            pl.BlockSpec((2, _RB, 16), lambda i: (0, i, 0)),
            pl.BlockSpec((2, _RB, _F), lambda i: (0, i, 0)),
            pl.BlockSpec((2, _RB, _NH), lambda i: (0, i, 0)),
            pl.BlockSpec((2, _RB, _NH), lambda i: (0, i, 0)),
            pl.BlockSpec((2, 1, 16), lambda i: (0, 0, 0)),
            pl.BlockSpec((2, _F), lambda i: (0, 0)),
            pl.BlockSpec((_NH, _F), lambda i: (0, 0)),
        ],
        out_specs=pl.BlockSpec((_RB, _F), lambda i: (i, 0)),
        out_shape=jax.ShapeDtypeStruct((_N, _F), jnp.float32),
    )(outu.reshape(2, _N, _F), den.reshape(2, _N, 16), Hc, Sc, Dc, Mc,
      jnp.stack([b1, b2]), Emat)
    return out


# speedup vs baseline: 205.1079x; 1.4271x over previous
import jax
import jax.numpy as jnp
from jax import lax
from jax.experimental import pallas as pl
from jax.experimental.pallas import tpu as pltpu
from jax.experimental.pallas import tpu_sc as plsc

_N = 10000
_E = 320000
_F = 128
_NH = 8
_NC = 16
_ALPHA = 0.5

_NSUB = 16
_LANES = 16
_EPT = _E // _NSUB
_B = 80
_NIT = _EPT // _B
_ZR = 624
_ZTAIL = _N - _NSUB * _ZR
_RB = 2000


def _pre_body(x_ref, w_ref, as_ref, ad_ref, h_ref, s_ref, d_ref, m_ref):
    xv = x_ref[...]
    h = lax.dot_general(xv, w_ref[0], (((1,), (1,)), ((), ())),
                        preferred_element_type=jnp.float32)
    s = jnp.dot(h, as_ref[0], preferred_element_type=jnp.float32)
    d = jnp.dot(h, ad_ref[0], preferred_element_type=jnp.float32)
    h_ref[0] = h
    s_ref[0] = s
    d_ref[0] = d
    b = jnp.max(s, axis=0, keepdims=True) + jnp.max(d, axis=0, keepdims=True)
    m = jnp.maximum(b, 0.2 * b)
    m_ref[0] = jnp.concatenate([m, m], axis=1)


def _sc_body(srcg_hbm, dstg_hbm, dsts_hbm, s_hbm, d_hbm, h_hbm, m_hbm,
             z128_hbm, z16_hbm, out_hbm, den_hbm,
             srcg_v0, srcg_v1, dstg_v0, dstg_v1, dsts_v0, dsts_v1,
             sg0, sg1, dg0, dg1, w2w0, w2w1, hrows0, hrows1, mvec_v,
             outacc, denacc,
             sem_i0, sem_i1, sem_g0, sem_g1, sem_s0, sem_s1):
    srcg_v = (srcg_v0, srcg_v1)
    dstg_v = (dstg_v0, dstg_v1)
    dsts_v = (dsts_v0, dsts_v1)
    sg = (sg0, sg1)
    dg = (dg0, dg1)
    w2w = (w2w0, w2w1)
    hrows = (hrows0, hrows1)
    sem_i = (sem_i0, sem_i1)
    sem_g = (sem_g0, sem_g1)
    sem_s = (sem_s0, sem_s1)

    c = lax.axis_index("c")
    t = lax.axis_index("s")
    rbase = t * _ZR

    pltpu.sync_copy(z128_hbm, outacc.at[pl.ds(rbase, _ZR)])
    pltpu.sync_copy(z16_hbm, denacc.at[pl.ds(rbase, _ZR)])

    @pl.when(t == _NSUB - 1)
    def _zero_tail():
        pltpu.sync_copy(z128_hbm.at[pl.ds(0, _ZTAIL)],
                        outacc.at[pl.ds(_NSUB * _ZR, _ZTAIL)])
        pltpu.sync_copy(z16_hbm.at[pl.ds(0, _ZTAIL)],
                        denacc.at[pl.ds(_NSUB * _ZR, _ZTAIL)])

    zv = jnp.zeros((_LANES,), jnp.float32)
    for bb in range(2):
        for i in range(_B):
            w2w[bb][i, :] = zv

    pltpu.sync_copy(m_hbm.at[pl.ds(c * 16, 16)], mvec_v)
    plsc.subcore_barrier()

    ebase0 = c * _E + t * _EPT

    def fire_idx(blk, b):
        eb = ebase0 + blk * _B
        pltpu.async_copy(srcg_hbm.at[pl.ds(eb, _B)], srcg_v[b], sem_i[b])
        pltpu.async_copy(dstg_hbm.at[pl.ds(eb, _B)], dstg_v[b], sem_i[b])
        pltpu.async_copy(dsts_hbm.at[pl.ds(eb, _B)], dsts_v[b], sem_i[b])

    def wait_idx(b):
        pltpu.make_async_copy(srcg_hbm.at[pl.ds(0, _B)], srcg_v[b], sem_i[b]).wait()
        pltpu.make_async_copy(dstg_hbm.at[pl.ds(0, _B)], dstg_v[b], sem_i[b]).wait()
        pltpu.make_async_copy(dsts_hbm.at[pl.ds(0, _B)], dsts_v[b], sem_i[b]).wait()

    def fire_gathers(b):
        pltpu.async_copy(s_hbm.at[srcg_v[b]], sg[b], sem_g[b])
        pltpu.async_copy(d_hbm.at[dstg_v[b]], dg[b], sem_g[b])
        pltpu.async_copy(h_hbm.at[srcg_v[b]], hrows[b], sem_g[b])

    def wait_gathers(b):
        pltpu.make_async_copy(s_hbm.at[srcg_v[b]], sg[b], sem_g[b]).wait()
        pltpu.make_async_copy(d_hbm.at[dstg_v[b]], dg[b], sem_g[b]).wait()
        pltpu.make_async_copy(h_hbm.at[srcg_v[b]], hrows[b], sem_g[b]).wait()

    def fire_scatter(b):
        pltpu.async_copy(hrows[b], outacc.at[dsts_v[b]], sem_s[b], add=True)
        pltpu.async_copy(w2w[b], denacc.at[dsts_v[b]], sem_s[b], add=True)

    def wait_scatter(b):
        pltpu.make_async_copy(hrows[b], outacc.at[dsts_v[b]], sem_s[b]).wait()
        pltpu.make_async_copy(w2w[b], denacc.at[dsts_v[b]], sem_s[b]).wait()

    def compute(b):
        @plsc.parallel_loop(0, _B // 2, unroll=4)
        def wbody(k):
            lanes = lax.iota(jnp.int32, 16)
            l8 = lanes // 8
            cols = lanes - 8 * l8
            mv = mvec_v[...]
            rows = 2 * k + l8
            g = (plsc.load_gather(sg[b], [rows, cols])
                 + plsc.load_gather(dg[b], [rows, cols]))
            wv = jnp.exp(jnp.maximum(g, 0.2 * g) - mv)
            plsc.store_scatter(w2w[b], [rows, cols], wv)

        @plsc.parallel_loop(0, _B, unroll=2)
        def mbody(e):
            wrow = w2w[b][e, :]
            for hh in range(_NH):
                hv = jnp.full((_LANES,), hh, dtype=jnp.int32)
                wb = wrow.at[hv].get(mode="promise_in_bounds")
                hrows[b][e, pl.ds(hh * 16, 16)] = (
                    hrows[b][e, pl.ds(hh * 16, 16)] * wb)

    pltpu.sync_copy(srcg_hbm.at[pl.ds(ebase0, _B)], srcg_v[0])
    pltpu.sync_copy(dstg_hbm.at[pl.ds(ebase0, _B)], dstg_v[0])
    pltpu.sync_copy(dsts_hbm.at[pl.ds(ebase0, _B)], dsts_v[0])
    fire_gathers(0)
    fire_idx(1, 1)

    def phase(blk, b):
        nb = 1 - b

        @pl.when(blk + 1 < _NIT)
        def _prefetch():
            wait_idx(nb)

            @pl.when(blk >= 1)
            def _protect():
                wait_scatter(nb)

            fire_gathers(nb)

        wait_gathers(b)
        compute(b)
        fire_scatter(b)

        @pl.when(blk + 2 < _NIT)
        def _next_idx():
            fire_idx(blk + 2, b)

    def outer(g, carry):
        phase(2 * g, 0)
        phase(2 * g + 1, 1)
        return carry

    lax.fori_loop(0, _NIT // 2, outer, 0)

    wait_scatter(0)
    wait_scatter(1)
    plsc.subcore_barrier()
    pltpu.sync_copy(outacc.at[pl.ds(rbase, _ZR)],
                    out_hbm.at[pl.ds(c * _N + rbase, _ZR)])
    pltpu.sync_copy(denacc.at[pl.ds(rbase, _ZR)],
                    den_hbm.at[pl.ds(c * _N + rbase, _ZR)])

    @pl.when(t == _NSUB - 1)
    def _write_tail():
        pltpu.sync_copy(outacc.at[pl.ds(_NSUB * _ZR, _ZTAIL)],
                        out_hbm.at[pl.ds(c * _N + _NSUB * _ZR, _ZTAIL)])
        pltpu.sync_copy(denacc.at[pl.ds(_NSUB * _ZR, _ZTAIL)],
                        den_hbm.at[pl.ds(c * _N + _NSUB * _ZR, _ZTAIL)])


def _post_body(u_ref, den_ref, h_ref, s_ref, d_ref, m_ref, b_ref, e_ref,
               out_ref):
    acc = None
    for dd in range(2):
        g = s_ref[dd] + d_ref[dd]
        m8 = m_ref[dd, 0:1, 0:8]
        ws = jnp.exp(jnp.maximum(g, 0.2 * g) - m8)
        wfull = jnp.dot(ws, e_ref[...], preferred_element_type=jnp.float32)
        dfull = jnp.dot(den_ref[dd, :, 0:8] + ws, e_ref[...],
                        preferred_element_type=jnp.float32)
        od = ((u_ref[dd] + wfull * h_ref[dd]) / (dfull + 1e-16)
              + b_ref[dd:dd + 1, :])
        w = (1.0 - _ALPHA) if dd == 0 else _ALPHA
        acc = w * od if acc is None else acc + w * od
    out_ref[...] = acc


def kernel(x, edge_index, W1, att_src1, att_dst1, b1, W2, att_src2,
           att_dst2, b2):
    src = edge_index[0]
    dst = edge_index[1]
    srcg = jnp.concatenate([src, dst + _N])
    dstg = jnp.concatenate([dst, src + _N])
    dsts = jnp.concatenate([dst, src])

    def blockdiag(att):
        a = att.reshape(_NH, _NC)
        return (jnp.eye(_NH, dtype=a.dtype)[:, None, :]
                * a[:, :, None]).reshape(_NH * _NC, _NH)

    Wstk = jnp.stack([W1, W2])
    AS = jnp.stack([blockdiag(att_src1), blockdiag(att_src2)])
    AD = jnp.stack([blockdiag(att_dst1), blockdiag(att_dst2)])
    Emat = jnp.repeat(jnp.eye(_NH, dtype=jnp.float32), _NC, axis=1)

    Hc, Sc, Dc, Mc = pl.pallas_call(
        _pre_body,
        grid=(2,),
        in_specs=[
            pl.BlockSpec((_N, _F), lambda d: (0, 0)),
            pl.BlockSpec((1, _F, _F), lambda d: (d, 0, 0)),
            pl.BlockSpec((1, _F, _NH), lambda d: (d, 0, 0)),
            pl.BlockSpec((1, _F, _NH), lambda d: (d, 0, 0)),
        ],
        out_specs=[
            pl.BlockSpec((1, _N, _F), lambda d: (d, 0, 0)),
            pl.BlockSpec((1, _N, _NH), lambda d: (d, 0, 0)),
            pl.BlockSpec((1, _N, _NH), lambda d: (d, 0, 0)),
            pl.BlockSpec((1, 1, 16), lambda d: (d, 0, 0)),
        ],
        out_shape=[
            jax.ShapeDtypeStruct((2, _N, _F), jnp.float32),
            jax.ShapeDtypeStruct((2, _N, _NH), jnp.float32),
            jax.ShapeDtypeStruct((2, _N, _NH), jnp.float32),
            jax.ShapeDtypeStruct((2, 1, 16), jnp.float32),
        ],
    )(x, Wstk, AS, AD)

    mesh = plsc.VectorSubcoreMesh(core_axis_name="c", subcore_axis_name="s")
    outu, den = pl.kernel(
        _sc_body,
        out_type=[
            jax.ShapeDtypeStruct((2 * _N, _F), jnp.float32),
            jax.ShapeDtypeStruct((2 * _N, 16), jnp.float32),
        ],
        mesh=mesh,
        compiler_params=pltpu.CompilerParams(use_tc_tiling_on_sc=False,
                                             needs_layout_passes=False),
        scratch_types=(
            [pltpu.VMEM((_B,), jnp.int32)] * 6
            + [pltpu.VMEM((_B, _NH), jnp.float32)] * 4
            + [pltpu.VMEM((_B, 16), jnp.float32)] * 2
            + [pltpu.VMEM((_B, _F), jnp.float32)] * 2
            + [pltpu.VMEM((16,), jnp.float32)]
            + [pltpu.VMEM_SHARED((_N, _F), jnp.float32),
               pltpu.VMEM_SHARED((_N, 16), jnp.float32)]
            + [pltpu.SemaphoreType.DMA] * 6
        ),
    )(srcg, dstg, dsts,
      Sc.reshape(2 * _N, _NH), Dc.reshape(2 * _N, _NH),
      Hc.reshape(2 * _N, _F), Mc.reshape(32),
      jnp.zeros((_ZR, _F), jnp.float32), jnp.zeros((_ZR, 16), jnp.float32))

    out = pl.pallas_call(
        _post_body,
        grid=(_N // _RB,),
        in_specs=[
            pl.BlockSpec((2, _RB, _F), lambda i: (0, i, 0)),
            pl.BlockSpec((2, _RB, 16), lambda i: (0, i, 0)),
            pl.BlockSpec((2, _RB, _F), lambda i: (0, i, 0)),
            pl.BlockSpec((2, _RB, _NH), lambda i: (0, i, 0)),
            pl.BlockSpec((2, _RB, _NH), lambda i: (0, i, 0)),
            pl.BlockSpec((2, 1, 16), lambda i: (0, 0, 0)),
            pl.BlockSpec((2, _F), lambda i: (0, 0)),
            pl.BlockSpec((_NH, _F), lambda i: (0, 0)),
        ],
        out_specs=pl.BlockSpec((_RB, _F), lambda i: (i, 0)),
        out_shape=jax.ShapeDtypeStruct((_N, _F), jnp.float32),
    )(outu.reshape(2, _N, _F), den.reshape(2, _N, 16), Hc, Sc, Dc, Mc,
      jnp.stack([b1, b2]), Emat)
    return out


# speedup vs baseline: 217.0877x; 1.0584x over previous
import jax
import jax.numpy as jnp
from jax import lax
from jax.experimental import pallas as pl
from jax.experimental.pallas import tpu as pltpu
from jax.experimental.pallas import tpu_sc as plsc

_N = 10000
_E = 320000
_F = 128
_NH = 8
_NC = 16
_ALPHA = 0.5

_NSUB = 16
_LANES = 16
_EPT = _E // _NSUB
_B = 80
_NIT = _EPT // _B
_ZR = 624
_ZTAIL = _N - _NSUB * _ZR
_RB = 2000


def _pre_body(x_ref, w_ref, as_ref, ad_ref, h_ref, s_ref, d_ref, m_ref):
    xv = x_ref[...]
    h = lax.dot_general(xv, w_ref[0], (((1,), (1,)), ((), ())),
                        preferred_element_type=jnp.float32)
    s = jnp.dot(h, as_ref[0], preferred_element_type=jnp.float32)
    d = jnp.dot(h, ad_ref[0], preferred_element_type=jnp.float32)
    h_ref[0] = h
    s_ref[0] = s
    d_ref[0] = d
    b = jnp.max(s, axis=0, keepdims=True) + jnp.max(d, axis=0, keepdims=True)
    m = jnp.maximum(b, 0.2 * b)
    m_ref[0] = jnp.concatenate([m, m], axis=1)


def _sc_body(srcg_hbm, dstg_hbm, dsts_hbm, s_hbm, d_hbm, h_hbm, m_hbm,
             z128_hbm, z16_hbm, out_hbm, den_hbm,
             srcg_v0, srcg_v1, dstg_v0, dstg_v1, dsts_v0, dsts_v1,
             sg0, sg1, dg0, dg1, w2w0, w2w1, hrows0, hrows1, mvec_v,
             outacc, denacc,
             sem_i0, sem_i1, sem_g0, sem_g1, sem_h0, sem_h1,
             sem_s0, sem_s1):
    srcg_v = (srcg_v0, srcg_v1)
    dstg_v = (dstg_v0, dstg_v1)
    dsts_v = (dsts_v0, dsts_v1)
    sg = (sg0, sg1)
    dg = (dg0, dg1)
    w2w = (w2w0, w2w1)
    hrows = (hrows0, hrows1)
    sem_i = (sem_i0, sem_i1)
    sem_g = (sem_g0, sem_g1)
    sem_h = (sem_h0, sem_h1)
    sem_s = (sem_s0, sem_s1)

    c = lax.axis_index("c")
    t = lax.axis_index("s")
    rbase = t * _ZR

    pltpu.sync_copy(z128_hbm, outacc.at[pl.ds(rbase, _ZR)])
    pltpu.sync_copy(z16_hbm, denacc.at[pl.ds(rbase, _ZR)])

    @pl.when(t == _NSUB - 1)
    def _zero_tail():
        pltpu.sync_copy(z128_hbm.at[pl.ds(0, _ZTAIL)],
                        outacc.at[pl.ds(_NSUB * _ZR, _ZTAIL)])
        pltpu.sync_copy(z16_hbm.at[pl.ds(0, _ZTAIL)],
                        denacc.at[pl.ds(_NSUB * _ZR, _ZTAIL)])

    zv = jnp.zeros((_LANES,), jnp.float32)
    for bb in range(2):
        for i in range(_B):
            w2w[bb][i, :] = zv

    pltpu.sync_copy(m_hbm.at[pl.ds(c * 16, 16)], mvec_v)
    plsc.subcore_barrier()

    ebase0 = c * _E + t * _EPT

    def fire_idx(blk, b):
        eb = ebase0 + blk * _B
        pltpu.async_copy(srcg_hbm.at[pl.ds(eb, _B)], srcg_v[b], sem_i[b])
        pltpu.async_copy(dstg_hbm.at[pl.ds(eb, _B)], dstg_v[b], sem_i[b])
        pltpu.async_copy(dsts_hbm.at[pl.ds(eb, _B)], dsts_v[b], sem_i[b])

    def wait_idx(b):
        pltpu.make_async_copy(srcg_hbm.at[pl.ds(0, _B)], srcg_v[b], sem_i[b]).wait()
        pltpu.make_async_copy(dstg_hbm.at[pl.ds(0, _B)], dstg_v[b], sem_i[b]).wait()
        pltpu.make_async_copy(dsts_hbm.at[pl.ds(0, _B)], dsts_v[b], sem_i[b]).wait()

    def fire_gathers(b):
        pltpu.async_copy(s_hbm.at[srcg_v[b]], sg[b], sem_g[b])
        pltpu.async_copy(d_hbm.at[dstg_v[b]], dg[b], sem_g[b])
        pltpu.async_copy(h_hbm.at[srcg_v[b]], hrows[b], sem_h[b])

    def wait_gathers_sd(b):
        pltpu.make_async_copy(s_hbm.at[srcg_v[b]], sg[b], sem_g[b]).wait()
        pltpu.make_async_copy(d_hbm.at[dstg_v[b]], dg[b], sem_g[b]).wait()

    def wait_gather_h(b):
        pltpu.make_async_copy(h_hbm.at[srcg_v[b]], hrows[b], sem_h[b]).wait()

    def fire_scatter(b):
        pltpu.async_copy(hrows[b], outacc.at[dsts_v[b]], sem_s[b], add=True)
        pltpu.async_copy(w2w[b], denacc.at[dsts_v[b]], sem_s[b], add=True)

    def wait_scatter(b):
        pltpu.make_async_copy(hrows[b], outacc.at[dsts_v[b]], sem_s[b]).wait()
        pltpu.make_async_copy(w2w[b], denacc.at[dsts_v[b]], sem_s[b]).wait()

    def compute_w(b):
        @plsc.parallel_loop(0, _B // 2, unroll=8)
        def wbody(k):
            lanes = lax.iota(jnp.int32, 16)
            l8 = lanes // 8
            cols = lanes - 8 * l8
            mv = mvec_v[...]
            rows = 2 * k + l8
            g = (plsc.load_gather(sg[b], [rows, cols])
                 + plsc.load_gather(dg[b], [rows, cols]))
            wv = jnp.exp(jnp.maximum(g, 0.2 * g) - mv)
            plsc.store_scatter(w2w[b], [rows, cols], wv)

    def compute_m(b):
        @plsc.parallel_loop(0, _B, unroll=4)
        def mbody(e):
            wrow = w2w[b][e, :]
            for hh in range(_NH):
                hv = jnp.full((_LANES,), hh, dtype=jnp.int32)
                wb = wrow.at[hv].get(mode="promise_in_bounds")
                hrows[b][e, pl.ds(hh * 16, 16)] = (
                    hrows[b][e, pl.ds(hh * 16, 16)] * wb)

    pltpu.sync_copy(srcg_hbm.at[pl.ds(ebase0, _B)], srcg_v[0])
    pltpu.sync_copy(dstg_hbm.at[pl.ds(ebase0, _B)], dstg_v[0])
    pltpu.sync_copy(dsts_hbm.at[pl.ds(ebase0, _B)], dsts_v[0])
    fire_gathers(0)
    fire_idx(1, 1)

    def phase(blk, b):
        nb = 1 - b

        @pl.when(blk + 1 < _NIT)
        def _prefetch():
            wait_idx(nb)

            @pl.when(blk >= 1)
            def _protect():
                wait_scatter(nb)

            fire_gathers(nb)

        wait_gathers_sd(b)
        compute_w(b)
        wait_gather_h(b)
        compute_m(b)
        fire_scatter(b)

        @pl.when(blk + 2 < _NIT)
        def _next_idx():
            fire_idx(blk + 2, b)

    def outer(g, carry):
        phase(2 * g, 0)
        phase(2 * g + 1, 1)
        return carry

    lax.fori_loop(0, _NIT // 2, outer, 0)

    wait_scatter(0)
    wait_scatter(1)
    plsc.subcore_barrier()
    pltpu.sync_copy(outacc.at[pl.ds(rbase, _ZR)],
                    out_hbm.at[pl.ds(c * _N + rbase, _ZR)])
    pltpu.sync_copy(denacc.at[pl.ds(rbase, _ZR)],
                    den_hbm.at[pl.ds(c * _N + rbase, _ZR)])

    @pl.when(t == _NSUB - 1)
    def _write_tail():
        pltpu.sync_copy(outacc.at[pl.ds(_NSUB * _ZR, _ZTAIL)],
                        out_hbm.at[pl.ds(c * _N + _NSUB * _ZR, _ZTAIL)])
        pltpu.sync_copy(denacc.at[pl.ds(_NSUB * _ZR, _ZTAIL)],
                        den_hbm.at[pl.ds(c * _N + _NSUB * _ZR, _ZTAIL)])


def _post_body(u_ref, den_ref, h_ref, s_ref, d_ref, m_ref, b_ref, e_ref,
               out_ref):
    acc = None
    for dd in range(2):
        g = s_ref[dd] + d_ref[dd]
        m8 = m_ref[dd, 0:1, 0:8]
        ws = jnp.exp(jnp.maximum(g, 0.2 * g) - m8)
        wfull = jnp.dot(ws, e_ref[...], preferred_element_type=jnp.float32)
        dfull = jnp.dot(den_ref[dd, :, 0:8] + ws, e_ref[...],
                        preferred_element_type=jnp.float32)
        od = ((u_ref[dd] + wfull * h_ref[dd]) / (dfull + 1e-16)
              + b_ref[dd:dd + 1, :])
        w = (1.0 - _ALPHA) if dd == 0 else _ALPHA
        acc = w * od if acc is None else acc + w * od
    out_ref[...] = acc


def kernel(x, edge_index, W1, att_src1, att_dst1, b1, W2, att_src2,
           att_dst2, b2):
    src = edge_index[0]
    dst = edge_index[1]
    srcg = jnp.concatenate([src, dst + _N])
    dstg = jnp.concatenate([dst, src + _N])
    dsts = jnp.concatenate([dst, src])

    def blockdiag(att):
        a = att.reshape(_NH, _NC)
        return (jnp.eye(_NH, dtype=a.dtype)[:, None, :]
                * a[:, :, None]).reshape(_NH * _NC, _NH)

    Wstk = jnp.stack([W1, W2])
    AS = jnp.stack([blockdiag(att_src1), blockdiag(att_src2)])
    AD = jnp.stack([blockdiag(att_dst1), blockdiag(att_dst2)])
    Emat = jnp.repeat(jnp.eye(_NH, dtype=jnp.float32), _NC, axis=1)

    Hc, Sc, Dc, Mc = pl.pallas_call(
        _pre_body,
        grid=(2,),
        in_specs=[
            pl.BlockSpec((_N, _F), lambda d: (0, 0)),
            pl.BlockSpec((1, _F, _F), lambda d: (d, 0, 0)),
            pl.BlockSpec((1, _F, _NH), lambda d: (d, 0, 0)),
            pl.BlockSpec((1, _F, _NH), lambda d: (d, 0, 0)),
        ],
        out_specs=[
            pl.BlockSpec((1, _N, _F), lambda d: (d, 0, 0)),
            pl.BlockSpec((1, _N, _NH), lambda d: (d, 0, 0)),
            pl.BlockSpec((1, _N, _NH), lambda d: (d, 0, 0)),
            pl.BlockSpec((1, 1, 16), lambda d: (d, 0, 0)),
        ],
        out_shape=[
            jax.ShapeDtypeStruct((2, _N, _F), jnp.float32),
            jax.ShapeDtypeStruct((2, _N, _NH), jnp.float32),
            jax.ShapeDtypeStruct((2, _N, _NH), jnp.float32),
            jax.ShapeDtypeStruct((2, 1, 16), jnp.float32),
        ],
    )(x, Wstk, AS, AD)

    mesh = plsc.VectorSubcoreMesh(core_axis_name="c", subcore_axis_name="s")
    outu, den = pl.kernel(
        _sc_body,
        out_type=[
            jax.ShapeDtypeStruct((2 * _N, _F), jnp.float32),
            jax.ShapeDtypeStruct((2 * _N, 16), jnp.float32),
        ],
        mesh=mesh,
        compiler_params=pltpu.CompilerParams(use_tc_tiling_on_sc=False,
                                             needs_layout_passes=False),
        scratch_types=(
            [pltpu.VMEM((_B,), jnp.int32)] * 6
            + [pltpu.VMEM((_B, _NH), jnp.float32)] * 4
            + [pltpu.VMEM((_B, 16), jnp.float32)] * 2
            + [pltpu.VMEM((_B, _F), jnp.float32)] * 2
            + [pltpu.VMEM((16,), jnp.float32)]
            + [pltpu.VMEM_SHARED((_N, _F), jnp.float32),
               pltpu.VMEM_SHARED((_N, 16), jnp.float32)]
            + [pltpu.SemaphoreType.DMA] * 8
        ),
    )(srcg, dstg, dsts,
      Sc.reshape(2 * _N, _NH), Dc.reshape(2 * _N, _NH),
      Hc.reshape(2 * _N, _F), Mc.reshape(32),
      jnp.zeros((_ZR, _F), jnp.float32), jnp.zeros((_ZR, 16), jnp.float32))

    out = pl.pallas_call(
        _post_body,
        grid=(_N // _RB,),
        in_specs=[
            pl.BlockSpec((2, _RB, _F), lambda i: (0, i, 0)),
            pl.BlockSpec((2, _RB, 16), lambda i: (0, i, 0)),
            pl.BlockSpec((2, _RB, _F), lambda i: (0, i, 0)),
            pl.BlockSpec((2, _RB, _NH), lambda i: (0, i, 0)),
            pl.BlockSpec((2, _RB, _NH), lambda i: (0, i, 0)),
            pl.BlockSpec((2, 1, 16), lambda i: (0, 0, 0)),
            pl.BlockSpec((2, _F), lambda i: (0, 0)),
            pl.BlockSpec((_NH, _F), lambda i: (0, 0)),
        ],
        out_specs=pl.BlockSpec((_RB, _F), lambda i: (i, 0)),
        out_shape=jax.ShapeDtypeStruct((_N, _F), jnp.float32),
    )(outu.reshape(2, _N, _F), den.reshape(2, _N, 16), Hc, Sc, Dc, Mc,
      jnp.stack([b1, b2]), Emat)
    return out


# speedup vs baseline: 232.5980x; 1.0714x over previous
import jax
import jax.numpy as jnp
from jax import lax
from jax.experimental import pallas as pl
from jax.experimental.pallas import tpu as pltpu
from jax.experimental.pallas import tpu_sc as plsc

_N = 10000
_E = 320000
_F = 128
_NH = 8
_NC = 16
_ALPHA = 0.5

_NSUB = 16
_LANES = 16
_B = 128
_EPTB = 156
_EPT = _EPTB * _B
_ZR = 624
_ZTAIL = _N - _NSUB * _ZR
_RB = 2000


def _pre_body(x_ref, w_ref, as_ref, ad_ref, h_ref, s_ref, d_ref, m_ref):
    xv = x_ref[...]
    h = lax.dot_general(xv, w_ref[0], (((1,), (1,)), ((), ())),
                        preferred_element_type=jnp.float32)
    s = jnp.dot(h, as_ref[0], preferred_element_type=jnp.float32)
    d = jnp.dot(h, ad_ref[0], preferred_element_type=jnp.float32)
    h_ref[0] = h
    s_ref[0] = s
    d_ref[0] = d
    b = jnp.max(s, axis=0, keepdims=True) + jnp.max(d, axis=0, keepdims=True)
    m = jnp.maximum(b, 0.2 * b)
    m_ref[0] = jnp.concatenate([m, m], axis=1)


def _sc_body(srcg_hbm, dstg_hbm, dsts_hbm, s_hbm, d_hbm, h_hbm, m_hbm,
             z128_hbm, z16_hbm, out_hbm, den_hbm,
             srcg_v0, srcg_v1, dstg_v0, dstg_v1, dsts_v0, dsts_v1,
             sg0, sg1, dg0, dg1, w2w0, w2w1, hrows0, hrows1, mvec_v,
             outacc, denacc,
             sem_i0, sem_i1, sem_g0, sem_g1, sem_h0, sem_h1,
             sem_s0, sem_s1):
    srcg_v = (srcg_v0, srcg_v1)
    dstg_v = (dstg_v0, dstg_v1)
    dsts_v = (dsts_v0, dsts_v1)
    sg = (sg0, sg1)
    dg = (dg0, dg1)
    w2w = (w2w0, w2w1)
    hrows = (hrows0, hrows1)
    sem_i = (sem_i0, sem_i1)
    sem_g = (sem_g0, sem_g1)
    sem_h = (sem_h0, sem_h1)
    sem_s = (sem_s0, sem_s1)

    c = lax.axis_index("c")
    t = lax.axis_index("s")
    rbase = t * _ZR

    pltpu.sync_copy(z128_hbm, outacc.at[pl.ds(rbase, _ZR)])
    pltpu.sync_copy(z16_hbm, denacc.at[pl.ds(rbase, _ZR)])

    @pl.when(t == _NSUB - 1)
    def _zero_tail():
        pltpu.sync_copy(z128_hbm.at[pl.ds(0, _ZTAIL)],
                        outacc.at[pl.ds(_NSUB * _ZR, _ZTAIL)])
        pltpu.sync_copy(z16_hbm.at[pl.ds(0, _ZTAIL)],
                        denacc.at[pl.ds(_NSUB * _ZR, _ZTAIL)])

    pltpu.sync_copy(m_hbm.at[pl.ds(c * 16, 16)], mvec_v)
    plsc.subcore_barrier()

    ebase0 = c * _E + t * _EPT
    nit = _EPTB + 4 * jnp.where(t == _NSUB - 1, 1, 0)

    def fire_idx(blk, b):
        eb = ebase0 + blk * _B
        pltpu.async_copy(srcg_hbm.at[pl.ds(eb, _B)], srcg_v[b], sem_i[b])
        pltpu.async_copy(dstg_hbm.at[pl.ds(eb, _B)], dstg_v[b], sem_i[b])
        pltpu.async_copy(dsts_hbm.at[pl.ds(eb, _B)], dsts_v[b], sem_i[b])

    def wait_idx(b):
        pltpu.make_async_copy(srcg_hbm.at[pl.ds(0, _B)], srcg_v[b], sem_i[b]).wait()
        pltpu.make_async_copy(dstg_hbm.at[pl.ds(0, _B)], dstg_v[b], sem_i[b]).wait()
        pltpu.make_async_copy(dsts_hbm.at[pl.ds(0, _B)], dsts_v[b], sem_i[b]).wait()

    def fire_gathers(b):
        pltpu.async_copy(s_hbm.at[srcg_v[b]], sg[b], sem_g[b])
        pltpu.async_copy(d_hbm.at[dstg_v[b]], dg[b], sem_g[b])
        pltpu.async_copy(h_hbm.at[srcg_v[b]], hrows[b], sem_h[b])

    def wait_gathers_sd(b):
        pltpu.make_async_copy(s_hbm.at[srcg_v[b]], sg[b], sem_g[b]).wait()
        pltpu.make_async_copy(d_hbm.at[dstg_v[b]], dg[b], sem_g[b]).wait()

    def wait_gather_h(b):
        pltpu.make_async_copy(h_hbm.at[srcg_v[b]], hrows[b], sem_h[b]).wait()

    def fire_scatter(b):
        pltpu.async_copy(hrows[b], outacc.at[dsts_v[b]], sem_s[b], add=True)
        pltpu.async_copy(w2w[b], denacc.at[dsts_v[b]], sem_s[b], add=True)

    def wait_scatter(b):
        pltpu.make_async_copy(hrows[b], outacc.at[dsts_v[b]], sem_s[b]).wait()
        pltpu.make_async_copy(w2w[b], denacc.at[dsts_v[b]], sem_s[b]).wait()

    def compute_w(b):
        @plsc.parallel_loop(0, _B // 2, unroll=8)
        def wbody(k):
            lanes = lax.iota(jnp.int32, 16)
            l8 = lanes // 8
            cols = lanes - 8 * l8
            mv = mvec_v[...]
            rows = 2 * k + l8
            g = (plsc.load_gather(sg[b], [rows, cols])
                 + plsc.load_gather(dg[b], [rows, cols]))
            wv = jnp.exp(jnp.maximum(g, 0.2 * g) - mv)
            plsc.store_scatter(w2w[b], [rows, cols], wv)

    def compute_m(b):
        @plsc.parallel_loop(0, _B // 2, unroll=4)
        def mbody(k):
            lanes = lax.iota(jnp.int32, 16)
            l8 = lanes // 8
            cols = lanes - 8 * l8
            rows = 2 * k + l8
            wpair = plsc.load_gather(w2w[b], [rows, cols])
            for ee in range(2):
                e = 2 * k + ee
                for hh in range(_NH):
                    hv = jnp.full((_LANES,), 8 * ee + hh, dtype=jnp.int32)
                    wb = wpair.at[hv].get(mode="promise_in_bounds")
                    hrows[b][e, pl.ds(hh * 16, 16)] = (
                        hrows[b][e, pl.ds(hh * 16, 16)] * wb)

    pltpu.sync_copy(srcg_hbm.at[pl.ds(ebase0, _B)], srcg_v[0])
    pltpu.sync_copy(dstg_hbm.at[pl.ds(ebase0, _B)], dstg_v[0])
    pltpu.sync_copy(dsts_hbm.at[pl.ds(ebase0, _B)], dsts_v[0])
    fire_gathers(0)
    fire_idx(1, 1)

    def phase(blk, b):
        nb = 1 - b

        @pl.when(blk + 1 < nit)
        def _prefetch():
            wait_idx(nb)

            @pl.when(blk >= 1)
            def _protect():
                wait_scatter(nb)

            fire_gathers(nb)

        wait_gathers_sd(b)
        compute_w(b)
        wait_gather_h(b)
        compute_m(b)
        fire_scatter(b)

        @pl.when(blk + 2 < nit)
        def _next_idx():
            fire_idx(blk + 2, b)

    def outer(g, carry):
        phase(2 * g, 0)
        phase(2 * g + 1, 1)
        return carry

    lax.fori_loop(0, nit // 2, outer, 0)

    wait_scatter(0)
    wait_scatter(1)
    plsc.subcore_barrier()
    pltpu.sync_copy(outacc.at[pl.ds(rbase, _ZR)],
                    out_hbm.at[pl.ds(c * _N + rbase, _ZR)])
    pltpu.sync_copy(denacc.at[pl.ds(rbase, _ZR)],
                    den_hbm.at[pl.ds(c * _N + rbase, _ZR)])

    @pl.when(t == _NSUB - 1)
    def _write_tail():
        pltpu.sync_copy(outacc.at[pl.ds(_NSUB * _ZR, _ZTAIL)],
                        out_hbm.at[pl.ds(c * _N + _NSUB * _ZR, _ZTAIL)])
        pltpu.sync_copy(denacc.at[pl.ds(_NSUB * _ZR, _ZTAIL)],
                        den_hbm.at[pl.ds(c * _N + _NSUB * _ZR, _ZTAIL)])


def _post_body(u_ref, den_ref, h_ref, s_ref, d_ref, m_ref, b_ref, e_ref,
               out_ref):
    acc = None
    for dd in range(2):
        g = s_ref[dd] + d_ref[dd]
        m8 = m_ref[dd, 0:1, 0:8]
        ws = jnp.exp(jnp.maximum(g, 0.2 * g) - m8)
        wfull = jnp.dot(ws, e_ref[...], preferred_element_type=jnp.float32)
        dfull = jnp.dot(den_ref[dd] + ws, e_ref[...],
                        preferred_element_type=jnp.float32)
        od = ((u_ref[dd] + wfull * h_ref[dd]) / (dfull + 1e-16)
              + b_ref[dd:dd + 1, :])
        w = (1.0 - _ALPHA) if dd == 0 else _ALPHA
        acc = w * od if acc is None else acc + w * od
    out_ref[...] = acc


def kernel(x, edge_index, W1, att_src1, att_dst1, b1, W2, att_src2,
           att_dst2, b2):
    src = edge_index[0]
    dst = edge_index[1]
    srcg = jnp.concatenate([src, dst + _N])
    dstg = jnp.concatenate([dst, src + _N])
    dsts = jnp.concatenate([dst, src])

    def blockdiag(att):
        a = att.reshape(_NH, _NC)
        return (jnp.eye(_NH, dtype=a.dtype)[:, None, :]
                * a[:, :, None]).reshape(_NH * _NC, _NH)

    Wstk = jnp.stack([W1, W2])
    AS = jnp.stack([blockdiag(att_src1), blockdiag(att_src2)])
    AD = jnp.stack([blockdiag(att_dst1), blockdiag(att_dst2)])
    Emat = jnp.repeat(jnp.eye(_NH, dtype=jnp.float32), _NC, axis=1)

    Hc, Sc, Dc, Mc = pl.pallas_call(
        _pre_body,
        grid=(2,),
        in_specs=[
            pl.BlockSpec((_N, _F), lambda d: (0, 0)),
            pl.BlockSpec((1, _F, _F), lambda d: (d, 0, 0)),
            pl.BlockSpec((1, _F, _NH), lambda d: (d, 0, 0)),
            pl.BlockSpec((1, _F, _NH), lambda d: (d, 0, 0)),
        ],
        out_specs=[
            pl.BlockSpec((1, _N, _F), lambda d: (d, 0, 0)),
            pl.BlockSpec((1, _N, _NH), lambda d: (d, 0, 0)),
            pl.BlockSpec((1, _N, _NH), lambda d: (d, 0, 0)),
            pl.BlockSpec((1, 1, 16), lambda d: (d, 0, 0)),
        ],
        out_shape=[
            jax.ShapeDtypeStruct((2, _N, _F), jnp.float32),
            jax.ShapeDtypeStruct((2, _N, _NH), jnp.float32),
            jax.ShapeDtypeStruct((2, _N, _NH), jnp.float32),
            jax.ShapeDtypeStruct((2, 1, 16), jnp.float32),
        ],
    )(x, Wstk, AS, AD)

    mesh = plsc.VectorSubcoreMesh(core_axis_name="c", subcore_axis_name="s")
    outu, den = pl.kernel(
        _sc_body,
        out_type=[
            jax.ShapeDtypeStruct((2 * _N, _F), jnp.float32),
            jax.ShapeDtypeStruct((2 * _N, _NH), jnp.float32),
        ],
        mesh=mesh,
        compiler_params=pltpu.CompilerParams(use_tc_tiling_on_sc=False,
                                             needs_layout_passes=False),
        scratch_types=(
            [pltpu.VMEM((_B,), jnp.int32)] * 6
            + [pltpu.VMEM((_B, _NH), jnp.float32)] * 4
            + [pltpu.VMEM((_B, _NH), jnp.float32)] * 2
            + [pltpu.VMEM((_B, _F), jnp.float32)] * 2
            + [pltpu.VMEM((16,), jnp.float32)]
            + [pltpu.VMEM_SHARED((_N, _F), jnp.float32),
               pltpu.VMEM_SHARED((_N, _NH), jnp.float32)]
            + [pltpu.SemaphoreType.DMA] * 8
        ),
    )(srcg, dstg, dsts,
      Sc.reshape(2 * _N, _NH), Dc.reshape(2 * _N, _NH),
      Hc.reshape(2 * _N, _F), Mc.reshape(32),
      jnp.zeros((_ZR, _F), jnp.float32), jnp.zeros((_ZR, _NH), jnp.float32))

    out = pl.pallas_call(
        _post_body,
        grid=(_N // _RB,),
        in_specs=[
            pl.BlockSpec((2, _RB, _F), lambda i: (0, i, 0)),
            pl.BlockSpec((2, _RB, _NH), lambda i: (0, i, 0)),
            pl.BlockSpec((2, _RB, _F), lambda i: (0, i, 0)),
            pl.BlockSpec((2, _RB, _NH), lambda i: (0, i, 0)),
            pl.BlockSpec((2, _RB, _NH), lambda i: (0, i, 0)),
            pl.BlockSpec((2, 1, 16), lambda i: (0, 0, 0)),
            pl.BlockSpec((2, _F), lambda i: (0, 0)),
            pl.BlockSpec((_NH, _F), lambda i: (0, 0)),
        ],
        out_specs=pl.BlockSpec((_RB, _F), lambda i: (i, 0)),
        out_shape=jax.ShapeDtypeStruct((_N, _F), jnp.float32),
    )(outu.reshape(2, _N, _F), den.reshape(2, _N, _NH), Hc, Sc, Dc, Mc,
      jnp.stack([b1, b2]), Emat)
    return out
